# baseline (speedup 1.0000x reference)
import functools

import jax
import jax.numpy as jnp
from jax import lax
from jax.experimental import pallas as pl
from jax.experimental.pallas import tpu as pltpu
from jax.experimental.pallas import tpu_sc as plsc

T, D, E, F, FS = 2048, 1024, 64, 512, 512
TT = 512
NCH = T // TT
BT = 64
S = 8192
G = S // BT
NSLOT = 2 * T


def _router_shared_step(x_ref, wr_ref, wgs_ref, wus_ref, wds_ref, wsg_ref,
                        out_ref, xbf_ref, topi_ref, topw_ref):
    for c in range(NCH):
        sl = pl.ds(c * TT, TT)
        x = x_ref[sl, :]
        xb = x.astype(jnp.bfloat16)
        xbf_ref[sl, :] = xb
        logits = jnp.dot(x, wr_ref[...], preferred_element_type=jnp.float32)
        m = jnp.max(logits, axis=-1, keepdims=True)
        ex = jnp.exp(logits - m)
        p = ex / jnp.sum(ex, axis=-1, keepdims=True)
        cols = jax.lax.broadcasted_iota(jnp.int32, p.shape, 1)
        m1 = jnp.max(p, axis=-1, keepdims=True)
        i1 = jnp.min(jnp.where(p == m1, cols, E), axis=-1, keepdims=True)
        pm = jnp.where(cols == i1, -1.0, p)
        m2 = jnp.max(pm, axis=-1, keepdims=True)
        i2 = jnp.min(jnp.where(pm == m2, cols, E), axis=-1, keepdims=True)
        s = m1 + m2
        topi_ref[sl, :] = jnp.concatenate([i1, i2], axis=1)
        topw_ref[sl, :] = jnp.concatenate([m1 / s, m2 / s], axis=1)
        g = jnp.dot(xb, wgs_ref[...].astype(jnp.bfloat16),
                    preferred_element_type=jnp.float32)
        u = jnp.dot(xb, wus_ref[...].astype(jnp.bfloat16),
                    preferred_element_type=jnp.float32)
        h = (jax.nn.silu(g) * u).astype(jnp.bfloat16)
        sh = jnp.dot(h, wds_ref[...].astype(jnp.bfloat16),
                     preferred_element_type=jnp.float32)
        sg = jax.nn.sigmoid(jnp.dot(x, wsg_ref[...],
                                    preferred_element_type=jnp.float32))
        out_ref[sl, :] = sg * sh


def _router_shared(x, Wr, Wg_s, Wu_s, Wd_s, Wsg):
    return pl.pallas_call(
        _router_shared_step,
        grid=(1,),
        in_specs=[
            pl.BlockSpec((T, D), lambda i: (0, 0)),
            pl.BlockSpec((D, E), lambda i: (0, 0)),
            pl.BlockSpec((D, FS), lambda i: (0, 0)),
            pl.BlockSpec((D, FS), lambda i: (0, 0)),
            pl.BlockSpec((FS, D), lambda i: (0, 0)),
            pl.BlockSpec((D, 1), lambda i: (0, 0)),
        ],
        out_specs=[
            pl.BlockSpec((T, D), lambda i: (0, 0)),
            pl.BlockSpec((T, D), lambda i: (0, 0)),
            pl.BlockSpec((T, 2), lambda i: (0, 0)),
            pl.BlockSpec((T, 2), lambda i: (0, 0)),
        ],
        out_shape=[
            jax.ShapeDtypeStruct((T, D), jnp.float32),
            jax.ShapeDtypeStruct((T, D), jnp.bfloat16),
            jax.ShapeDtypeStruct((T, 2), jnp.int32),
            jax.ShapeDtypeStruct((T, 2), jnp.float32),
        ],
    )(x, Wr, Wg_s, Wu_s, Wd_s, Wsg)


def _ggemm_step(teid_ref, nt_ref, tok_ref, sw_ref, xbf_ref, wg_ref, wu_ref,
                wd_ref, hg_ref):
    g = pl.program_id(0)

    @pl.when(g < nt_ref[0])
    def _():
        onehot = (jax.lax.broadcasted_iota(jnp.int32, (BT, T), 1)
                  == tok_ref[...]).astype(jnp.bfloat16)
        xg = jnp.dot(onehot, xbf_ref[...],
                     preferred_element_type=jnp.float32).astype(jnp.bfloat16)
        gg = jnp.dot(xg, wg_ref[0].astype(jnp.bfloat16),
                     preferred_element_type=jnp.float32)
        uu = jnp.dot(xg, wu_ref[0].astype(jnp.bfloat16),
                     preferred_element_type=jnp.float32)
        h = (jax.nn.silu(gg) * uu).astype(jnp.bfloat16)
        hd = jnp.dot(h, wd_ref[0].astype(jnp.bfloat16),
                     preferred_element_type=jnp.float32)
        hg_ref[...] = hd * sw_ref[...]


def _grouped_gemm(tile_eid, n_tiles, sorted_tok, sorted_w, xbf,
                  Wg_e, Wu_e, Wd_e):
    grid_spec = pltpu.PrefetchScalarGridSpec(
        num_scalar_prefetch=2,
        grid=(G,),
        in_specs=[
            pl.BlockSpec((BT, 1), lambda g, teid, nt: (g, 0)),
            pl.BlockSpec((BT, 1), lambda g, teid, nt: (g, 0)),
            pl.BlockSpec((T, D), lambda g, teid, nt: (0, 0)),
            pl.BlockSpec((1, D, F), lambda g, teid, nt: (teid[g], 0, 0)),
            pl.BlockSpec((1, D, F), lambda g, teid, nt: (teid[g], 0, 0)),
            pl.BlockSpec((1, F, D), lambda g, teid, nt: (teid[g], 0, 0)),
        ],
        out_specs=pl.BlockSpec((BT, D), lambda g, teid, nt: (g, 0)),
    )
    return pl.pallas_call(
        _ggemm_step,
        grid_spec=grid_spec,
        out_shape=jax.ShapeDtypeStruct((S, D), jnp.float32),
        compiler_params=pltpu.CompilerParams(
            dimension_semantics=("arbitrary",),
        ),
    )(tile_eid, n_tiles, sorted_tok.reshape(S, 1),
      sorted_w.reshape(S, 1), xbf, Wg_e, Wu_e, Wd_e)


NW4 = 32
TPW = T // NW4
CH = 16


def _combine_body(oinit_hbm, hg_hbm, inv_hbm, out_hbm,
                  idx_v, rows_v, acc_v, sem):
    cid = lax.axis_index("c")
    sid = lax.axis_index("s")
    wid = sid * 2 + cid
    for c in range(TPW // CH):
        tb = wid * TPW + c * CH
        pltpu.sync_copy(inv_hbm.at[pl.ds(2 * tb, 2 * CH)], idx_v)
        pltpu.async_copy(hg_hbm.at[idx_v], rows_v, sem).wait()
        pltpu.sync_copy(oinit_hbm.at[pl.ds(tb, CH)], acc_v)

        for j in range(CH):
            def jq(q, c_, j=j):
                qq = q * 16
                a = acc_v[j, pl.ds(qq, 16)]
                r0 = rows_v[2 * j, pl.ds(qq, 16)]
                r1 = rows_v[2 * j + 1, pl.ds(qq, 16)]
                acc_v[j, pl.ds(qq, 16)] = a + r0 + r1
                return c_
            lax.fori_loop(0, D // 16, jq, jnp.int32(0))
        pltpu.sync_copy(acc_v, out_hbm.at[pl.ds(tb, CH)])


def _combine(out_init, hg, inv):
    mesh = plsc.VectorSubcoreMesh(core_axis_name="c", subcore_axis_name="s", num_cores=2, num_subcores=16)
    f = pl.kernel(
        _combine_body,
        compiler_params=pltpu.CompilerParams(needs_layout_passes=False),
        out_type=jax.ShapeDtypeStruct((T, D), jnp.float32),
        mesh=mesh,
        scratch_types=[
            pltpu.VMEM((2 * CH,), jnp.int32),
            pltpu.VMEM((2 * CH, D), jnp.float32),
            pltpu.VMEM((CH, D), jnp.float32),
            pltpu.SemaphoreType.DMA,
        ],
    )
    return f(out_init, hg, inv)


def _dispatch_meta_jnp(topi, topw):
    eflat = topi.reshape(-1)
    wflat = topw.reshape(-1)
    onehot = (eflat[:, None] == jnp.arange(E, dtype=jnp.int32)[None, :])
    csum = jnp.cumsum(onehot.astype(jnp.int32), axis=0)
    rank = jnp.take_along_axis(csum, eflat[:, None], axis=1)[:, 0] - 1
    counts = csum[-1]
    ntiles_e = (counts + BT - 1) // BT
    tile_off = jnp.cumsum(ntiles_e) - ntiles_e
    n_tiles = jnp.sum(ntiles_e).astype(jnp.int32)
    pos = tile_off[eflat] * BT + rank
    sorted_tok = jnp.zeros((S,), jnp.int32).at[pos].set(
        jnp.arange(2 * T, dtype=jnp.int32) // 2)
    sorted_w = jnp.zeros((S,), jnp.float32).at[pos].set(wflat)
    g_ar = jnp.arange(G, dtype=jnp.int32)
    te_full = jnp.searchsorted(tile_off + ntiles_e, g_ar, side='right'
                               ).astype(jnp.int32)
    e_last = te_full[jnp.maximum(n_tiles - 1, 0)]
    tile_eid = jnp.where(g_ar < n_tiles, te_full, e_last)
    nt16 = jnp.full((16,), n_tiles, jnp.int32)
    return sorted_tok, sorted_w, tile_eid, nt16, pos


def kernel(hidden_states, Wr, Wg_e, Wu_e, Wd_e, Wg_s, Wu_s, Wd_s, Wsg):
    x = hidden_states.reshape(-1, hidden_states.shape[-1])
    out_init, xbf, topi, topw = _router_shared(x, Wr, Wg_s, Wu_s, Wd_s, Wsg)
    stok, sw, teid, nt, inv = _dispatch_meta_jnp(topi, topw)
    hg = _grouped_gemm(teid, nt, stok, sw, xbf, Wg_e, Wu_e, Wd_e)
    out = _combine(out_init, hg, inv)
    return out.reshape(hidden_states.shape)
